# Initial kernel scaffold; baseline (speedup 1.0000x reference)
import jax
import jax.numpy as jnp
from jax.experimental import pallas as pl


def kernel(input, emb_weight, clause_idx, clause_sign):
    raise NotImplementedError("write your pallas kernel here")


# speedup vs baseline: 1.3491x; 1.3491x over previous
import jax
import jax.numpy as jnp
from jax import lax
from jax.experimental import pallas as pl
from jax.experimental.pallas import tpu as pltpu
from jax.experimental.pallas import tpu_sc as plsc

B = 1024
NV = 2000
NC = 8000
K = 3

LANES = 16
NUM_CORES = 2
NUM_SUBCORES = 16
NW = NUM_CORES * NUM_SUBCORES
BPW = B // NW
NG = NC // LANES
CPR = NV // LANES


def _sc_body(inp_hbm, emb_hbm, i0_hbm, i1_hbm, i2_hbm, s0_hbm, s1_hbm, s2_hbm,
             out_hbm,
             idxv, tbl, i0v, i1v, i2v, s0v, s1v, s2v, accs, outv, sem):
    w = lax.axis_index("s") * NUM_CORES + lax.axis_index("c")
    base = w * BPW

    pltpu.sync_copy(inp_hbm.at[pl.ds(base, BPW)], idxv)
    gather_rows = pltpu.async_copy(emb_hbm.at[idxv], tbl, sem)
    pltpu.sync_copy(i0_hbm, i0v)
    pltpu.sync_copy(i1_hbm, i1v)
    pltpu.sync_copy(i2_hbm, i2v)
    pltpu.sync_copy(s0_hbm, s0v)
    pltpu.sync_copy(s1_hbm, s1v)
    pltpu.sync_copy(s2_hbm, s2v)
    gather_rows.wait()

    for b in range(BPW):
        def _sig(c, _, b=b):
            sl = pl.ds(c * LANES, LANES)
            z = tbl[b, sl]
            tbl[b, sl] = 1.0 / (1.0 + jnp.exp(-z))
            return _
        lax.fori_loop(0, CPR, _sig, None)

    one = jnp.full((LANES,), 1.0, jnp.float32)
    for b in range(BPW):
        accs[pl.ds(b * LANES, LANES)] = one

    bvecs = [jnp.full((LANES,), b, jnp.int32) for b in range(BPW)]

    def _grp(g, _):
        sl = pl.ds(g * LANES, LANES)
        id0 = i0v[sl]
        id1 = i1v[sl]
        id2 = i2v[sl]
        sg0 = s0v[sl]
        sg1 = s1v[sl]
        sg2 = s2v[sl]
        a0 = 0.5 + 0.5 * sg0
        a1 = 0.5 + 0.5 * sg1
        a2 = 0.5 + 0.5 * sg2
        for b in range(BPW):
            l0 = plsc.load_gather(tbl, [bvecs[b], id0])
            l1 = plsc.load_gather(tbl, [bvecs[b], id1])
            l2 = plsc.load_gather(tbl, [bvecs[b], id2])
            t = (a0 - sg0 * l0) * (a1 - sg1 * l1) * (a2 - sg2 * l2)
            ab = pl.ds(b * LANES, LANES)
            accs[ab] = accs[ab] * (1.0 - t)
        return _

    lax.fori_loop(0, NG, _grp, None)

    lane = lax.iota(jnp.int32, LANES)
    for half in range(2):
        bidx = lane * LANES + half * (LANES * LANES)
        p = plsc.load_gather(accs, [bidx])
        for l in range(1, LANES):
            p = p * plsc.load_gather(accs, [bidx + l])
        outv[pl.ds(half * LANES, LANES)] = p

    pltpu.sync_copy(outv, out_hbm.at[pl.ds(base, BPW)])


def kernel(input, emb_weight, clause_idx, clause_sign):
    inp = input.astype(jnp.int32)
    ci = clause_idx.astype(jnp.int32)
    i0, i1, i2 = ci[:, 0], ci[:, 1], ci[:, 2]
    cs = clause_sign.astype(jnp.float32)
    s0, s1, s2 = cs[:, 0], cs[:, 1], cs[:, 2]

    mesh = plsc.VectorSubcoreMesh(
        core_axis_name="c", subcore_axis_name="s",
        num_cores=NUM_CORES, num_subcores=NUM_SUBCORES)
    f = pl.kernel(
        _sc_body,
        out_type=jax.ShapeDtypeStruct((B,), jnp.float32),
        mesh=mesh,
        compiler_params=pltpu.CompilerParams(
            use_tc_tiling_on_sc=False, needs_layout_passes=False),
        scratch_types=[
            pltpu.VMEM((BPW,), jnp.int32),
            pltpu.VMEM((BPW, NV), jnp.float32),
            pltpu.VMEM((NC,), jnp.int32),
            pltpu.VMEM((NC,), jnp.int32),
            pltpu.VMEM((NC,), jnp.int32),
            pltpu.VMEM((NC,), jnp.float32),
            pltpu.VMEM((NC,), jnp.float32),
            pltpu.VMEM((NC,), jnp.float32),
            pltpu.VMEM((BPW * LANES,), jnp.float32),
            pltpu.VMEM((BPW,), jnp.float32),
            pltpu.SemaphoreType.DMA,
        ],
    )
    return f(inp, emb_weight.astype(jnp.float32), i0, i1, i2, s0, s1, s2)


# speedup vs baseline: 4.6669x; 3.4592x over previous
import jax
import jax.numpy as jnp
from jax import lax
from jax.experimental import pallas as pl
from jax.experimental.pallas import tpu as pltpu
from jax.experimental.pallas import tpu_sc as plsc

B = 1024
NV = 2000
NC = 8000
K = 3

LANES = 16
NUM_CORES = 2
NUM_SUBCORES = 16
NW = NUM_CORES * NUM_SUBCORES
BPW = B // NW
NG = NC // LANES
CPR = NV // LANES
HB = BPW // 2


def _sc_body(inp_hbm, emb_hbm, i0_hbm, i1_hbm, i2_hbm, s0_hbm, s1_hbm, s2_hbm,
             out_hbm,
             idxv, tbl, i0v, i1v, i2v, s0v, s1v, s2v, accs, outv, sem, sem2):
    w = lax.axis_index("s") * NUM_CORES + lax.axis_index("c")
    base = w * BPW

    pltpu.sync_copy(inp_hbm.at[pl.ds(base, BPW)], idxv)
    tbl_cp = pltpu.async_copy(emb_hbm.at[idxv], tbl, sem)
    cps = [pltpu.async_copy(src, dst, sem2)
           for src, dst in ((i0_hbm, i0v), (i1_hbm, i1v), (i2_hbm, i2v),
                            (s0_hbm, s0v), (s1_hbm, s1v), (s2_hbm, s2v))]
    for cp in cps:
        cp.wait()
    tbl_cp.wait()

    def _sig(c, carry):
        sl = pl.ds(c * LANES, LANES)
        for b in range(BPW):
            z = tbl[b, sl]
            tbl[b, sl] = 1.0 / (1.0 + jnp.exp(-z))
        return carry
    lax.fori_loop(0, CPR, _sig, None)

    one = jnp.full((LANES,), 1.0, jnp.float32)
    for half in range(2):
        def _grp(g, acc_c, half=half):
            sl = pl.ds(g * LANES, LANES)
            id0 = i0v[sl]
            id1 = i1v[sl]
            id2 = i2v[sl]
            sg0 = s0v[sl]
            sg1 = s1v[sl]
            sg2 = s2v[sl]
            a0 = 0.5 + 0.5 * sg0
            a1 = 0.5 + 0.5 * sg1
            a2 = 0.5 + 0.5 * sg2
            nxt = []
            for j in range(HB):
                row = tbl.at[half * HB + j]
                l0 = plsc.load_gather(row, [id0])
                l1 = plsc.load_gather(row, [id1])
                l2 = plsc.load_gather(row, [id2])
                t = (a0 - sg0 * l0) * (a1 - sg1 * l1) * (a2 - sg2 * l2)
                nxt.append(acc_c[j] * (1.0 - t))
            return tuple(nxt)
        fin = lax.fori_loop(0, NG, _grp, (one,) * HB)
        for j in range(HB):
            accs[pl.ds((half * HB + j) * LANES, LANES)] = fin[j]

    lane = lax.iota(jnp.int32, LANES)
    for half in range(2):
        bidx = lane * LANES + half * (LANES * LANES)
        p = plsc.load_gather(accs, [bidx])
        for l in range(1, LANES):
            p = p * plsc.load_gather(accs, [bidx + l])
        outv[pl.ds(half * LANES, LANES)] = p

    pltpu.sync_copy(outv, out_hbm.at[pl.ds(base, BPW)])


def kernel(input, emb_weight, clause_idx, clause_sign):
    inp = input.astype(jnp.int32)
    ci = clause_idx.astype(jnp.int32)
    i0, i1, i2 = ci[:, 0], ci[:, 1], ci[:, 2]
    cs = clause_sign.astype(jnp.float32)
    s0, s1, s2 = cs[:, 0], cs[:, 1], cs[:, 2]

    mesh = plsc.VectorSubcoreMesh(
        core_axis_name="c", subcore_axis_name="s",
        num_cores=NUM_CORES, num_subcores=NUM_SUBCORES)
    f = pl.kernel(
        _sc_body,
        out_type=jax.ShapeDtypeStruct((B,), jnp.float32),
        mesh=mesh,
        compiler_params=pltpu.CompilerParams(
            use_tc_tiling_on_sc=False, needs_layout_passes=False),
        scratch_types=[
            pltpu.VMEM((BPW,), jnp.int32),
            pltpu.VMEM((BPW, NV), jnp.float32),
            pltpu.VMEM((NC,), jnp.int32),
            pltpu.VMEM((NC,), jnp.int32),
            pltpu.VMEM((NC,), jnp.int32),
            pltpu.VMEM((NC,), jnp.float32),
            pltpu.VMEM((NC,), jnp.float32),
            pltpu.VMEM((NC,), jnp.float32),
            pltpu.VMEM((BPW * LANES,), jnp.float32),
            pltpu.VMEM((BPW,), jnp.float32),
            pltpu.SemaphoreType.DMA,
            pltpu.SemaphoreType.DMA,
        ],
    )
    return f(inp, emb_weight.astype(jnp.float32), i0, i1, i2, s0, s1, s2)
